# baseline (speedup 1.0000x reference)
import functools

import jax
import jax.numpy as jnp
from jax import lax
from jax.experimental import pallas as pl
from jax.experimental.pallas import tpu as pltpu
from jax.experimental.pallas import tpu_sc as plsc

D_MODEL = 32
N_BATCH = 16384
N_SEQ = 50
N_TOKENS = N_BATCH * N_SEQ

_TBLK = 8192
_QBLK = _TBLK // 4
_NBLK = 123
_VPAD = _NBLK * _TBLK

_NC = 2
_NS = 16
_NW = _NC * _NS
_PER_W = N_TOKENS // _NW
_GRP = 512
_GPW = _PER_W // _GRP

_mesh = plsc.VectorSubcoreMesh(core_axis_name="c", subcore_axis_name="s")


def _transpose_body(wt_ref, out_ref):
    eye = jnp.eye(D_MODEL, dtype=jnp.float32)
    pieces = []
    for q in range(4):
        blkq = wt_ref[:, q * _QBLK:(q + 1) * _QBLK]
        pieces.append(
            lax.dot_general(blkq, eye, (((0,), (0,)), ((), ())),
                            precision=lax.Precision.HIGHEST,
                            preferred_element_type=jnp.float32))
    out_ref[...] = jnp.concatenate(pieces, axis=1)


def _rm_table(weight):
    out = pl.pallas_call(
        _transpose_body,
        grid=(_NBLK,),
        in_specs=[pl.BlockSpec((32, _TBLK), lambda i: (0, i))],
        out_specs=pl.BlockSpec((_QBLK, 128), lambda i: (i, 0)),
        out_shape=jax.ShapeDtypeStruct((_VPAD // 4, 128), jnp.float32),
    )(weight.T)
    return out.reshape(_VPAD, D_MODEL)


@functools.partial(
    pl.kernel,
    out_type=jax.ShapeDtypeStruct((N_TOKENS, D_MODEL), jnp.float32),
    mesh=_mesh,
    scratch_types=[
        pltpu.VMEM((_PER_W,), jnp.int32),
        pltpu.VMEM((4, 128), jnp.int32),
        pltpu.VMEM((4, 128), jnp.int32),
        pltpu.VMEM((4, 128), jnp.int32),
        pltpu.VMEM((_GRP, D_MODEL), jnp.float32),
        pltpu.VMEM((_GRP, D_MODEL), jnp.float32),
        pltpu.SemaphoreType.DMA,
        pltpu.SemaphoreType.DMA,
        pltpu.SemaphoreType.DMA,
        pltpu.SemaphoreType.DMA,
    ],
    compiler_params=pltpu.CompilerParams(use_tc_tiling_on_sc=False),
)
def _sc_gather(idx_hbm, table_hbm, out_hbm,
               idx_v, jpat, oidx0, oidx1, rows0, rows1,
               g0, g1, s0, s1):
    wid = lax.axis_index("s") * _NC + lax.axis_index("c")
    base = wid * _PER_W
    gbase0 = wid * _GPW
    oidx = (oidx0, oidx1)
    rows = (rows0, rows1)
    gsem = (g0, g1)
    ssem = (s0, s1)
    iota = lax.iota(jnp.int32, 16)

    pltpu.sync_copy(idx_hbm.at[pl.ds(base, _PER_W)], idx_v)

    def remap(k, carry):
        sl = pl.ds(k * 16, 16)
        v = idx_v[sl]
        rem = lax.bitwise_and(v, 8191)
        q = lax.shift_right_logical(rem, 11)
        r = lax.bitwise_and(rem, 2047)
        idx_v[sl] = (v - rem) + lax.shift_left(r, 2) + q
        return carry

    lax.fori_loop(0, _PER_W // 16, remap, 0)

    for k in range(4):
        for m in range(8):
            p = iota + (k * 128 + m * 16)
            jpat[k, pl.ds(m * 16, 16)] = lax.bitwise_or(
                lax.shift_left(lax.bitwise_and(p, 127), 2),
                lax.shift_right_logical(p, 7))

    def prep_oidx(g, b):
        off = (gbase0 + g) * _GRP
        for k in range(4):
            for m in range(8):
                sl = pl.ds(m * 16, 16)
                oidx[b][k, sl] = jpat[k, sl] + off

    def start_gather(g, b):
        pltpu.async_copy(
            table_hbm.at[idx_v.at[pl.ds(g * _GRP, _GRP)]], rows[b], gsem[b])

    def wait_gather(g, b):
        pltpu.make_async_copy(
            table_hbm.at[idx_v.at[pl.ds(g * _GRP, _GRP)]],
            rows[b], gsem[b]).wait()

    def start_scatters(b):
        for k in range(4):
            pltpu.async_copy(
                rows[b].at[pl.ds(k * 128, 128)],
                out_hbm.at[oidx[b].at[k]], ssem[b])

    def wait_scatters(b):
        for k in range(4):
            pltpu.make_async_copy(
                rows[b].at[pl.ds(k * 128, 128)],
                out_hbm.at[oidx[b].at[k]], ssem[b]).wait()

    prep_oidx(0, 0)
    start_gather(0, 0)

    def body(j, carry):
        g_even = 2 * j

        @pl.when(j > 0)
        def _():
            wait_scatters(1)
        prep_oidx(g_even + 1, 1)
        start_gather(g_even + 1, 1)

        wait_gather(g_even, 0)
        start_scatters(0)

        @pl.when(j < _GPW // 2 - 1)
        def _():
            wait_scatters(0)
            prep_oidx(g_even + 2, 0)
            start_gather(g_even + 2, 0)

        wait_gather(g_even + 1, 1)
        start_scatters(1)
        return carry

    lax.fori_loop(0, _GPW // 2, body, 0)
    wait_scatters(0)
    wait_scatters(1)


def _deint_body(g_ref, out_ref):
    eye = jnp.eye(D_MODEL, dtype=jnp.float32)
    pieces = []
    for gr in range(4):
        for q in range(4):
            blk = g_ref[0, gr * 128:(gr + 1) * 128,
                        q * D_MODEL:(q + 1) * D_MODEL]
            pieces.append(
                lax.dot_general(eye, blk, (((1,), (1,)), ((), ())),
                                precision=lax.Precision.HIGHEST,
                                preferred_element_type=jnp.float32))
    out_ref[0] = jnp.concatenate(pieces, axis=1)


def _deinterleave(g):
    return pl.pallas_call(
        _deint_body,
        grid=(N_SEQ, N_BATCH // 2048),
        in_specs=[pl.BlockSpec((1, 512, 128), lambda s, b: (s, b, 0))],
        out_specs=pl.BlockSpec((1, D_MODEL, 2048), lambda s, b: (s, 0, b)),
        out_shape=jax.ShapeDtypeStruct((N_SEQ, D_MODEL, N_BATCH),
                                       jnp.float32),
    )(g)


def kernel(token_ids, weight):
    idx_sm = token_ids.T.reshape(-1).astype(jnp.int32)
    g = _sc_gather(idx_sm, _rm_table(weight))
    out = _deinterleave(g.reshape(N_SEQ, N_BATCH // 4, 128))
    return out.transpose(2, 0, 1)


# speedup vs baseline: 1.0394x; 1.0394x over previous
import functools

import jax
import jax.numpy as jnp
from jax import lax
from jax.experimental import pallas as pl
from jax.experimental.pallas import tpu as pltpu
from jax.experimental.pallas import tpu_sc as plsc

D_MODEL = 32
N_BATCH = 16384
N_SEQ = 50
N_TOKENS = N_BATCH * N_SEQ

_NC = 2
_NS = 16
_NW = _NC * _NS
_PER_W = N_TOKENS // _NW
_GRP = 512
_GPW = _PER_W // _GRP

_mesh = plsc.VectorSubcoreMesh(core_axis_name="c", subcore_axis_name="s")


@functools.partial(
    pl.kernel,
    out_type=jax.ShapeDtypeStruct((N_TOKENS, D_MODEL), jnp.float32),
    mesh=_mesh,
    scratch_types=[
        pltpu.VMEM((_PER_W,), jnp.int32),
        pltpu.VMEM((4, 128), jnp.int32),
        pltpu.VMEM((4, 128), jnp.int32),
        pltpu.VMEM((4, 128), jnp.int32),
        pltpu.VMEM((_GRP, D_MODEL), jnp.float32),
        pltpu.VMEM((_GRP, D_MODEL), jnp.float32),
        pltpu.SemaphoreType.DMA,
        pltpu.SemaphoreType.DMA,
        pltpu.SemaphoreType.DMA,
        pltpu.SemaphoreType.DMA,
    ],
    compiler_params=pltpu.CompilerParams(use_tc_tiling_on_sc=False),
)
def _sc_gather(idx_hbm, table_hbm, out_hbm,
               idx_v, jpat, oidx0, oidx1, rows0, rows1,
               g0, g1, s0, s1):
    wid = lax.axis_index("s") * _NC + lax.axis_index("c")
    base = wid * _PER_W
    gbase0 = wid * _GPW
    oidx = (oidx0, oidx1)
    rows = (rows0, rows1)
    gsem = (g0, g1)
    ssem = (s0, s1)
    iota = lax.iota(jnp.int32, 16)

    pltpu.sync_copy(idx_hbm.at[pl.ds(base, _PER_W)], idx_v)

    for k in range(4):
        for m in range(8):
            p = iota + (k * 128 + m * 16)
            jpat[k, pl.ds(m * 16, 16)] = lax.bitwise_or(
                lax.shift_left(lax.bitwise_and(p, 127), 2),
                lax.shift_right_logical(p, 7))

    def prep_oidx(g, b):
        off = (gbase0 + g) * _GRP
        for k in range(4):
            for m in range(8):
                sl = pl.ds(m * 16, 16)
                oidx[b][k, sl] = jpat[k, sl] + off

    def start_gather(g, b):
        pltpu.async_copy(
            table_hbm.at[idx_v.at[pl.ds(g * _GRP, _GRP)]], rows[b], gsem[b])

    def wait_gather(g, b):
        pltpu.make_async_copy(
            table_hbm.at[idx_v.at[pl.ds(g * _GRP, _GRP)]],
            rows[b], gsem[b]).wait()

    def start_scatters(b):
        for k in range(4):
            pltpu.async_copy(
                rows[b].at[pl.ds(k * 128, 128)],
                out_hbm.at[oidx[b].at[k]], ssem[b])

    def wait_scatters(b):
        for k in range(4):
            pltpu.make_async_copy(
                rows[b].at[pl.ds(k * 128, 128)],
                out_hbm.at[oidx[b].at[k]], ssem[b]).wait()

    prep_oidx(0, 0)
    start_gather(0, 0)

    def body(j, carry):
        g_even = 2 * j

        @pl.when(j > 0)
        def _():
            wait_scatters(1)
        prep_oidx(g_even + 1, 1)
        start_gather(g_even + 1, 1)

        wait_gather(g_even, 0)
        start_scatters(0)

        @pl.when(j < _GPW // 2 - 1)
        def _():
            wait_scatters(0)
            prep_oidx(g_even + 2, 0)
            start_gather(g_even + 2, 0)

        wait_gather(g_even + 1, 1)
        start_scatters(1)
        return carry

    lax.fori_loop(0, _GPW // 2, body, 0)
    wait_scatters(0)
    wait_scatters(1)


def _deint_body(g_ref, out_ref):
    eye = jnp.eye(D_MODEL, dtype=jnp.float32)
    pieces = []
    for gr in range(4):
        for q in range(4):
            blk = g_ref[0, gr * 128:(gr + 1) * 128,
                        q * D_MODEL:(q + 1) * D_MODEL]
            pieces.append(
                lax.dot_general(eye, blk, (((1,), (1,)), ((), ())),
                                precision=lax.Precision.HIGHEST,
                                preferred_element_type=jnp.float32))
    out_ref[0] = jnp.concatenate(pieces, axis=1)


def _deinterleave(g):
    return pl.pallas_call(
        _deint_body,
        grid=(N_SEQ, N_BATCH // 2048),
        in_specs=[pl.BlockSpec((1, 512, 128), lambda s, b: (s, b, 0))],
        out_specs=pl.BlockSpec((1, D_MODEL, 2048), lambda s, b: (s, 0, b)),
        out_shape=jax.ShapeDtypeStruct((N_SEQ, D_MODEL, N_BATCH),
                                       jnp.float32),
    )(g)


def kernel(token_ids, weight):
    idx_sm = token_ids.T.reshape(-1).astype(jnp.int32)
    g = _sc_gather(idx_sm, weight)
    out = _deinterleave(g.reshape(N_SEQ, N_BATCH // 4, 128))
    return out.transpose(2, 0, 1)
